# baseline (speedup 1.0000x reference)
import functools

import jax
import jax.numpy as jnp
from jax import lax
from jax.experimental import pallas as pl
from jax.experimental.pallas import tpu as pltpu
from jax.experimental.pallas import tpu_sc as plsc

_B, _D = 16384, 2048
_BSC = 12288
_NC, _NS = 2, 16
_NW = _NC * _NS
_RPW = _BSC // _NW
_R = 8
_NCHUNK = _RPW // _R


def _sc_body(x_hbm, perm_hbm, out_hbm, perm_v, in0, in1, o0, o1,
             sin0, sin1, sout0, sout1):
    wid = lax.axis_index("s") * _NC + lax.axis_index("c")
    base = wid * _RPW
    pltpu.sync_copy(perm_hbm, perm_v)

    bufs = ((in0, o0, sin0, sout0), (in1, o1, sin1, sout1))

    def in_copy(c, b):
        row = base + c * _R
        return pltpu.make_async_copy(
            x_hbm.at[pl.ds(row, _R)], bufs[b][0], bufs[b][2])

    def out_copy(c, b):
        row = base + c * _R
        return pltpu.make_async_copy(
            bufs[b][1], out_hbm.at[pl.ds(row, _R)], bufs[b][3])

    in_copy(0, 0).start()

    def outer_body(c2, _):
        for b in range(2):
            c = c2 * 2 + b
            inb, outb = bufs[b][0], bufs[b][1]

            @pl.when(c + 1 < _NCHUNK)
            def _prefetch():
                in_copy(c + 1, 1 - b).start()

            in_copy(c, b).wait()

            @pl.when(c >= 2)
            def _drain():
                out_copy(c - 2, b).wait()

            @plsc.parallel_loop(0, _D // 16, unroll=8)
            def _gather(j):
                jb = j * 16
                pv = perm_v[pl.ds(jb, 16)]
                for r in range(_R):
                    rr = jnp.full((16,), r, jnp.int32)
                    outb[r, pl.ds(jb, 16)] = plsc.load_gather(inb, [rr, pv])

            out_copy(c, b).start()
        return 0

    lax.fori_loop(0, _NCHUNK // 2, outer_body, 0)
    out_copy(_NCHUNK - 2, 0).wait()
    out_copy(_NCHUNK - 1, 1).wait()


def _sc_permute(x_top, perm):
    mesh = plsc.VectorSubcoreMesh(core_axis_name="c", subcore_axis_name="s")
    f = functools.partial(
        pl.kernel,
        mesh=mesh,
        compiler_params=pltpu.CompilerParams(needs_layout_passes=False),
        out_type=jax.ShapeDtypeStruct((_BSC, _D), jnp.float32),
        scratch_types=[
            pltpu.VMEM((_D,), jnp.int32),
            pltpu.VMEM((_R, _D), jnp.float32),
            pltpu.VMEM((_R, _D), jnp.float32),
            pltpu.VMEM((_R, _D), jnp.float32),
            pltpu.VMEM((_R, _D), jnp.float32),
            pltpu.SemaphoreType.DMA,
            pltpu.SemaphoreType.DMA,
            pltpu.SemaphoreType.DMA,
            pltpu.SemaphoreType.DMA,
        ],
    )(_sc_body)
    return f(x_top, perm)


def _tc_body(x_ref, out_ref):
    out_ref[...] = x_ref[...]


def _tc_copy(x_bot):
    rows = 256
    n = x_bot.shape[0]
    return pl.pallas_call(
        _tc_body,
        grid=(n // rows,),
        in_specs=[pl.BlockSpec((rows, _D), lambda i: (i, 0))],
        out_specs=pl.BlockSpec((rows, _D), lambda i: (i, 0)),
        out_shape=jax.ShapeDtypeStruct((n, _D), jnp.float32),
    )(x_bot)


@jax.jit
def _permute(x, perm):
    top = _sc_permute(x[:_BSC], perm)
    bot = _tc_copy(x[_BSC:])
    return jnp.concatenate([top, bot], axis=0)


def kernel(x, perm):
    out = _permute(x, perm)
    logdet = jnp.zeros((_B,), x.dtype)
    return (out, logdet)


# speedup vs baseline: 2.3087x; 2.3087x over previous
import functools

import jax
import jax.numpy as jnp
from jax import lax
from jax.experimental import pallas as pl
from jax.experimental.pallas import tpu as pltpu
from jax.experimental.pallas import tpu_sc as plsc

_B, _D = 16384, 2048
_NC, _NS = 2, 16
_NW = _NC * _NS
_RPW = _B // _NW
_R = 8
_NCHUNK = _RPW // _R


def _sc_body(x_hbm, perm_hbm, out_hbm, perm_v, in0, in1, o0, o1,
             sin0, sin1, sout0, sout1):
    wid = lax.axis_index("s") * _NC + lax.axis_index("c")
    base = wid * _RPW
    pltpu.sync_copy(perm_hbm, perm_v)

    bufs = ((in0, o0, sin0, sout0), (in1, o1, sin1, sout1))

    def in_copy(c, b):
        row = base + c * _R
        return pltpu.make_async_copy(
            x_hbm.at[pl.ds(row, _R)], bufs[b][0], bufs[b][2])

    def out_copy(c, b):
        row = base + c * _R
        return pltpu.make_async_copy(
            bufs[b][1], out_hbm.at[pl.ds(row, _R)], bufs[b][3])

    in_copy(0, 0).start()

    def outer_body(c2, _):
        for b in range(2):
            c = c2 * 2 + b
            inb, outb = bufs[b][0], bufs[b][1]

            @pl.when(c + 1 < _NCHUNK)
            def _prefetch():
                in_copy(c + 1, 1 - b).start()

            in_copy(c, b).wait()

            @pl.when(c >= 2)
            def _drain():
                out_copy(c - 2, b).wait()

            @plsc.parallel_loop(0, _D // 16, unroll=8)
            def _gather(j):
                jb = j * 16
                pv = perm_v[pl.ds(jb, 16)]
                for r in range(_R):
                    rr = jnp.full((16,), r, jnp.int32)
                    outb[r, pl.ds(jb, 16)] = plsc.load_gather(inb, [rr, pv])

            out_copy(c, b).start()
        return 0

    lax.fori_loop(0, _NCHUNK // 2, outer_body, 0)
    out_copy(_NCHUNK - 2, 0).wait()
    out_copy(_NCHUNK - 1, 1).wait()


@jax.jit
def _permute(x, perm):
    mesh = plsc.VectorSubcoreMesh(core_axis_name="c", subcore_axis_name="s")
    f = functools.partial(
        pl.kernel,
        mesh=mesh,
        compiler_params=pltpu.CompilerParams(needs_layout_passes=False),
        out_type=jax.ShapeDtypeStruct((_B, _D), jnp.float32),
        scratch_types=[
            pltpu.VMEM((_D,), jnp.int32),
            pltpu.VMEM((_R, _D), jnp.float32),
            pltpu.VMEM((_R, _D), jnp.float32),
            pltpu.VMEM((_R, _D), jnp.float32),
            pltpu.VMEM((_R, _D), jnp.float32),
            pltpu.SemaphoreType.DMA,
            pltpu.SemaphoreType.DMA,
            pltpu.SemaphoreType.DMA,
            pltpu.SemaphoreType.DMA,
        ],
    )(_sc_body)
    return f(x, perm)


def kernel(x, perm):
    out = _permute(x, perm)
    logdet = jnp.zeros((_B,), x.dtype)
    return (out, logdet)


# speedup vs baseline: 2.3575x; 1.0211x over previous
import functools

import jax
import jax.numpy as jnp
from jax import lax
from jax.experimental import pallas as pl
from jax.experimental.pallas import tpu as pltpu
from jax.experimental.pallas import tpu_sc as plsc

_B, _D = 16384, 2048
_NC, _NS = 2, 16
_NW = _NC * _NS
_RPW = _B // _NW
_RIN = 16
_RO = 8
_NIN = _RPW // _RIN
_NOUT = _RPW // _RO


def _sc_body(x_hbm, perm_hbm, out_hbm, perm_v, in0, in1, o0, o1,
             sin0, sin1, sout0, sout1):
    wid = lax.axis_index("s") * _NC + lax.axis_index("c")
    base = wid * _RPW
    pltpu.sync_copy(perm_hbm, perm_v)

    ins = ((in0, sin0), (in1, sin1))
    outs = ((o0, sout0), (o1, sout1))

    def in_copy(k, b):
        row = base + k * _RIN
        return pltpu.make_async_copy(
            x_hbm.at[pl.ds(row, _RIN)], ins[b][0], ins[b][1])

    def out_copy(c, h):
        row = base + c * _RO
        return pltpu.make_async_copy(
            outs[h][0], out_hbm.at[pl.ds(row, _RO)], outs[h][1])

    in_copy(0, 0).start()

    def outer_body(k2, _):
        for kk in range(2):
            k = k2 * 2 + kk
            inb = ins[kk][0]

            @pl.when(k + 1 < _NIN)
            def _prefetch():
                in_copy(k + 1, 1 - kk).start()

            in_copy(k, kk).wait()

            for h in range(2):
                c = k * 2 + h
                outb = outs[h][0]

                @pl.when(c >= 2)
                def _drain():
                    out_copy(c - 2, h).wait()

                @plsc.parallel_loop(0, _D // 16, unroll=8)
                def _gather(j):
                    jb = j * 16
                    pv = perm_v[pl.ds(jb, 16)]
                    for r in range(_RO):
                        rr = jnp.full((16,), h * _RO + r, jnp.int32)
                        outb[r, pl.ds(jb, 16)] = plsc.load_gather(
                            inb, [rr, pv])

                out_copy(c, h).start()
        return 0

    lax.fori_loop(0, _NIN // 2, outer_body, 0)
    out_copy(_NOUT - 2, 0).wait()
    out_copy(_NOUT - 1, 1).wait()


@jax.jit
def _permute(x, perm):
    mesh = plsc.VectorSubcoreMesh(core_axis_name="c", subcore_axis_name="s")
    f = functools.partial(
        pl.kernel,
        mesh=mesh,
        compiler_params=pltpu.CompilerParams(needs_layout_passes=False),
        out_type=jax.ShapeDtypeStruct((_B, _D), jnp.float32),
        scratch_types=[
            pltpu.VMEM((_D,), jnp.int32),
            pltpu.VMEM((_RIN, _D), jnp.float32),
            pltpu.VMEM((_RIN, _D), jnp.float32),
            pltpu.VMEM((_RO, _D), jnp.float32),
            pltpu.VMEM((_RO, _D), jnp.float32),
            pltpu.SemaphoreType.DMA,
            pltpu.SemaphoreType.DMA,
            pltpu.SemaphoreType.DMA,
            pltpu.SemaphoreType.DMA,
        ],
    )(_sc_body)
    return f(x, perm)


def kernel(x, perm):
    out = _permute(x, perm)
    logdet = jnp.zeros((_B,), x.dtype)
    return (out, logdet)
